# baseline (speedup 1.0000x reference)
import functools

import jax
import jax.numpy as jnp
from jax import lax
from jax.experimental import pallas as pl
from jax.experimental.pallas import tpu as pltpu
from jax.experimental.pallas import tpu_sc as plsc

_B = 16384
_D = 64
_E = 100000
_L = 16
_NC = 2
_NS = 16
_NW = _NC * _NS
_PW = _B // _NW
_S = 4096
_NSTR = _B // _S
_CCH = 128


def _phase1_body(headT, tailT, relT, rinvT,
                 h_idx, r_idx, t_idx,
                 a_out, b_out, c_out, d_out, e_out, f_out,
                 plane_v, idx0_v, idx1_v, val0_v, val1_v,
                 sem_i, sem_o):
  wid = lax.axis_index("s") * _NC + lax.axis_index("c")
  idx_bufs = (idx0_v, idx1_v)
  val_bufs = (val0_v, val1_v)

  def gather_role(d, idx_hbm, out_hbm):
    pltpu.async_copy(idx_hbm.at[pl.ds(0, _S)], idx_bufs[0], sem_i)
    out_cps = []
    for s in range(_NSTR):
      idx_v = idx_bufs[s % 2]
      val_v = val_bufs[s % 2]
      pltpu.make_async_copy(idx_hbm.at[pl.ds(s * _S, _S)], idx_v,
                            sem_i).wait()
      if s + 1 < _NSTR:
        pltpu.async_copy(idx_hbm.at[pl.ds((s + 1) * _S, _S)],
                         idx_bufs[(s + 1) % 2], sem_i)
      if s >= 2:
        out_cps[s - 2].wait()

      def gbody(g, carry):
        for u in range(16):
          sl = pl.ds((g * 16 + u) * _L, _L)
          val_v[sl] = plsc.load_gather(plane_v, [idx_v[sl]])
        return carry

      lax.fori_loop(0, _S // (16 * _L), gbody, 0)
      out_cps.append(
          pltpu.async_copy(val_v, out_hbm.at[d, pl.ds(s * _S, _S)], sem_o))
    for c in out_cps[max(0, _NSTR - 2):]:
      c.wait()

  for rnd in range(8):
    tbl = (headT, headT, tailT, tailT, relT, relT, rinvT, rinvT)[rnd]
    d = (rnd % 2) * 32 + wid
    pltpu.sync_copy(tbl.at[d], plane_v)
    if rnd < 2:
      gather_role(d, h_idx, a_out)
      gather_role(d, t_idx, d_out)
    elif rnd < 4:
      gather_role(d, t_idx, c_out)
      gather_role(d, h_idx, f_out)
    elif rnd < 6:
      gather_role(d, r_idx, b_out)
    else:
      gather_role(d, r_idx, e_out)


def _phase2_tc(a_ref, b_ref, c_ref, d_ref, e_ref, f_ref, out_ref):
  prod = a_ref[...] * b_ref[...] * c_ref[...] \
      + d_ref[...] * e_ref[...] * f_ref[...]
  out_ref[...] = 0.5 * jnp.sum(prod, axis=0)


@jax.jit
def _simple_score(h_idx, r_idx, t_idx, headT, tailT, relT, rinvT):
  mesh = plsc.VectorSubcoreMesh(
      core_axis_name="c", subcore_axis_name="s",
      num_cores=_NC, num_subcores=_NS)
  gmat = jax.ShapeDtypeStruct((_D, _B), jnp.float32)
  params = pltpu.CompilerParams(needs_layout_passes=False)
  p1 = functools.partial(
      pl.kernel,
      out_type=(gmat,) * 6,
      mesh=mesh,
      compiler_params=params,
      scratch_types=[
          pltpu.VMEM((_E,), jnp.float32),
          pltpu.VMEM((_S,), jnp.int32),
          pltpu.VMEM((_S,), jnp.int32),
          pltpu.VMEM((_S,), jnp.float32),
          pltpu.VMEM((_S,), jnp.float32),
          pltpu.SemaphoreType.DMA,
          pltpu.SemaphoreType.DMA,
      ],
  )(_phase1_body)
  a, b, c, d, e, f = p1(headT, tailT, relT, rinvT, h_idx, r_idx, t_idx)

  blk = 2048
  in_spec = pl.BlockSpec((_D, blk), lambda i: (0, i))
  p2 = pl.pallas_call(
      _phase2_tc,
      out_shape=jax.ShapeDtypeStruct((_B,), jnp.float32),
      grid=(_B // blk,),
      in_specs=[in_spec] * 6,
      out_specs=pl.BlockSpec((blk,), lambda i: (i,)),
  )
  return p2(a, b, c, d, e, f)


def kernel(sample, head_embedding, tail_embedding, relation_embedding,
           relation_inverse_embedding):
  sample = sample.astype(jnp.int32)
  h_idx = sample[:, 0]
  r_idx = sample[:, 1]
  t_idx = sample[:, 2]
  return _simple_score(h_idx, r_idx, t_idx,
                       head_embedding.T, tail_embedding.T,
                       relation_embedding.T, relation_inverse_embedding.T)
